# baseline (speedup 1.0000x reference)
import functools
import jax
import jax.numpy as jnp
from jax import lax
from jax.experimental import pallas as pl
from jax.experimental.pallas import tpu as pltpu

EMBED = 128
NREL_PAD = 1024
BLOCK = 1024


def _tc_body(h_ref, t_ref, idx_ref, tab_ref, o_ref):
    idx = idx_ref[0, 0]
    rel_iota = lax.broadcasted_iota(jnp.int32, (BLOCK, NREL_PAD), 1)
    onehot = jnp.where(rel_iota == idx[:, None], 1.0, 0.0).astype(jnp.bfloat16)
    r_emb = jnp.dot(onehot, tab_ref[...],
                    preferred_element_type=jnp.float32)
    p = h_ref[...] * t_ref[...]
    o_ref[0, 0] = jnp.sum(p * r_emb, axis=1)


def kernel(h_emb, t_emb, r_type, relation_embed):
    batch = h_emb.shape[0]
    idx = r_type.astype(jnp.int32)
    tab = jnp.zeros((NREL_PAD, EMBED), jnp.bfloat16).at[:1000].set(
        relation_embed.astype(jnp.bfloat16))
    grid = (batch // BLOCK,)
    return pl.pallas_call(
        _tc_body,
        grid=grid,
        in_specs=[
            pl.BlockSpec((BLOCK, EMBED), lambda i: (i, 0)),
            pl.BlockSpec((BLOCK, EMBED), lambda i: (i, 0)),
            pl.BlockSpec((1, 1, BLOCK), lambda i: (i, 0, 0)),
            pl.BlockSpec((NREL_PAD, EMBED), lambda i: (0, 0)),
        ],
        out_specs=pl.BlockSpec((1, 1, BLOCK), lambda i: (i, 0, 0)),
        out_shape=jax.ShapeDtypeStruct((batch // BLOCK, 1, BLOCK), jnp.float32),
    )(h_emb, t_emb, idx.reshape(batch // BLOCK, 1, BLOCK), tab).reshape(batch)
